# baseline (speedup 1.0000x reference)
import jax
import jax.numpy as jnp
from jax.experimental import pallas as pl
from jax.experimental.pallas import tpu as pltpu

_N = 20000
_G = 64
_B = 2560
_NB = 8

_VAR_X = 0.1
_VAR_Y = 0.1
_VAR_W = 0.2
_VAR_H = 0.2
_ALPHA = 1.0
_THR = 0.5
_NEG2POS = 6
_MIN_NEG = 10
_MAX_BACK_CF = 0.5
_NEG_LAMBDA = 1.0

_F32_INF_BITS = 0x7F800000


def _kth_smallest_stats(arr, k):
    bits = jax.lax.bitcast_convert_type(arr, jnp.int32)

    def step(_, lohi):
        lo, hi = lohi
        mid = lo + (hi - lo) // 2
        c = jnp.sum((bits <= mid).astype(jnp.int32))
        ge = c >= k
        return jnp.where(ge, lo, mid + 1), jnp.where(ge, mid, hi)

    lo, _ = jax.lax.fori_loop(
        0, 31, step, (jnp.int32(0), jnp.int32(_F32_INF_BITS))
    )
    t = jax.lax.bitcast_convert_type(lo, jnp.float32)
    lt = arr < t
    c_lt = jnp.sum(lt.astype(jnp.int32))
    slog = jnp.sum(jnp.where(lt, jnp.log(jnp.where(lt, arr, 1.0)), 0.0))
    return t, c_lt, slog


def _transpose_mxu(blk, nrows):
    del nrows
    return jnp.transpose(blk)


def _loss_body(
    pbd_ref, pcf_ref, gt_ref, awh_ref, axy_ref, out_ref,
    negv_ref, cf1v_ref, cnt_ref, fac_ref,
):
    i = pl.program_id(0)

    @pl.when(i == 0)
    def _init():
        cnt_ref[0] = 0
        cnt_ref[1] = 0
        fac_ref[0] = 0.0
        fac_ref[1] = 0.0

    gxmin = gt_ref[:, 1:2]
    gymin = gt_ref[:, 2:3]
    gw = gt_ref[:, 3:4]
    gh = gt_ref[:, 4:5]
    gxmax = gxmin + gw
    gymax = gymin + gh
    gcx = gxmin + gw * 0.5
    gcy = gymin + gh * 0.5

    tpbd = _transpose_mxu(pbd_ref[:, :], 4)
    tcf = _transpose_mxu(pcf_ref[:, :], 2)
    tawh = _transpose_mxu(awh_ref[:, :], 4)
    taxy = _transpose_mxu(axy_ref[:, :], 4)
    pbd = [tpbd[j : j + 1, :] for j in range(4)]
    cf0 = tcf[0:1, :]
    cf1 = tcf[1:2, :]
    acx = tawh[0:1, :]
    acy = tawh[1:2, :]
    aw = tawh[2:3, :]
    ah = tawh[3:4, :]
    axmin = taxy[0:1, :]
    aymin = taxy[1:2, :]
    axmax = taxy[2:3, :]
    aymax = taxy[3:4, :]
    lane = jax.lax.broadcasted_iota(jnp.int32, (1, _B), 1)
    valid = (lane + i * _B) < _N

    iw = jnp.maximum(jnp.minimum(axmax, gxmax) - jnp.maximum(axmin, gxmin), 0.0)
    ih = jnp.maximum(jnp.minimum(aymax, gymax) - jnp.maximum(aymin, gymin), 0.0)
    inter = iw * ih
    area_a = (axmax - axmin) * (aymax - aymin)
    area_b = gw * gh
    union = (area_a + area_b) - inter
    pos = ((inter + inter) >= union) & valid
    posf = pos.astype(jnp.float32)
    pos_per_anchor = jnp.sum(posf, axis=0, keepdims=True)
    neg_row = (pos_per_anchor == 0.0) & valid

    inv_aw = (1.0 / _VAR_X) / aw
    inv_ah = (1.0 / _VAR_Y) / ah
    law = jnp.log(aw) * (1.0 / _VAR_W)
    lah = jnp.log(ah) * (1.0 / _VAR_H)
    lgw = jnp.log(gw) * (1.0 / _VAR_W)
    lgh = jnp.log(gh) * (1.0 / _VAR_H)

    def _sl1(d):
        ad = jnp.abs(d)
        m = jnp.minimum(ad, 1.0)
        return m * (ad - 0.5 * m)

    s = _sl1(pbd[0] - (gcx - acx) * inv_aw)
    s = s + _sl1(pbd[1] - (gcy - acy) * inv_ah)
    s = s + _sl1(pbd[2] - (lgw - law))
    s = s + _sl1(pbd[3] - (lgh - lah))
    loc = jnp.sum(jnp.where(pos, s, 0.0))

    cnt_ref[0] += jnp.sum(pos_per_anchor).astype(jnp.int32)
    cnt_ref[1] += jnp.sum(neg_row.astype(jnp.int32))
    fac_ref[0] += loc
    fac_ref[1] += jnp.sum(
        pos_per_anchor * jnp.log(jnp.where(valid, cf0, 1.0))
    )

    negv_ref[pl.ds(i, 1), :] = jnp.where(neg_row, cf1, jnp.inf)
    cf1v_ref[pl.ds(i, 1), :] = jnp.where(valid, cf1, jnp.inf)

    @pl.when(i == _NB - 1)
    def _finalize():
        num_pos = cnt_ref[0]
        num_neg = cnt_ref[1]
        loc_loss = fac_ref[0]
        pos_cf_sum = fac_ref[1]

        neg_arr = negv_ref[:, :]
        c05 = jnp.sum((neg_arr < _MAX_BACK_CF).astype(jnp.int32))
        n_hard = jnp.minimum(jnp.maximum(num_pos * _NEG2POS, _MIN_NEG), num_neg)
        n_m = jnp.minimum(n_hard, c05)
        t, c_lt, slog = _kth_smallest_stats(neg_arr, n_m)
        t_safe = jnp.where(n_m > 0, t, 1.0)
        s_mined = slog + (n_m - c_lt).astype(jnp.float32) * jnp.log(t_safe)
        neg_cf_loss = jnp.where(
            n_m == 0,
            jnp.float32(0.0),
            -s_mined / jnp.maximum(n_m, 1).astype(jnp.float32) * _NEG_LAMBDA,
        )
        num_pos_f = jnp.maximum(num_pos, 1).astype(jnp.float32)
        loss = (
            _ALPHA * loc_loss / num_pos_f - pos_cf_sum / num_pos_f + neg_cf_loss
        )
        out_ref[:, :] = jnp.broadcast_to(loss, (1, 1))

        @pl.when(num_pos == 0)
        def _no_positives():
            t0, c0, slog0 = _kth_smallest_stats(cf1v_ref[:, :], _MIN_NEG)
            s0 = slog0 + (_MIN_NEG - c0).astype(jnp.float32) * jnp.log(t0)
            out_ref[:, :] = jnp.broadcast_to(
                -s0 / float(_MIN_NEG) * _NEG_LAMBDA, (1, 1)
            )


def kernel(pred_box_delt, pred_CF, GT_box_wh, Anchor_box_wh, Anchor_box_xy):
    out = pl.pallas_call(
        _loss_body,
        grid=(_NB,),
        in_specs=[
            pl.BlockSpec((_B, 4), lambda i: (i, 0)),
            pl.BlockSpec((_B, 2), lambda i: (i, 0)),
            pl.BlockSpec((_G, 5), lambda i: (0, 0)),
            pl.BlockSpec((_B, 4), lambda i: (i, 0)),
            pl.BlockSpec((_B, 4), lambda i: (i, 0)),
        ],
        out_specs=pl.BlockSpec((1, 1), lambda i: (0, 0)),
        out_shape=jax.ShapeDtypeStruct((1, 1), jnp.float32),
        scratch_shapes=[
            pltpu.VMEM((_NB, _B), jnp.float32),
            pltpu.VMEM((_NB, _B), jnp.float32),
            pltpu.SMEM((2,), jnp.int32),
            pltpu.SMEM((2,), jnp.float32),
        ],
    )(pred_box_delt, pred_CF, GT_box_wh, Anchor_box_wh, Anchor_box_xy)
    return out[0, 0]


# speedup vs baseline: 2.1209x; 2.1209x over previous
import jax
import jax.numpy as jnp
from jax.experimental import pallas as pl
from jax.experimental.pallas import tpu as pltpu

_N = 20000
_G = 64
_B = 5120
_NB = 4
_NPAD = _B * _NB

_VAR_X = 0.1
_VAR_Y = 0.1
_VAR_W = 0.2
_VAR_H = 0.2
_ALPHA = 1.0
_THR = 0.5
_NEG2POS = 6
_MIN_NEG = 10
_MAX_BACK_CF = 0.5
_NEG_LAMBDA = 1.0

_F32_INF_BITS = 0x7F800000


def _kth_smallest_stats(arr, k):
    bits = jax.lax.bitcast_convert_type(arr, jnp.int32)

    def step(_, lohi):
        lo, hi = lohi
        mid = lo + (hi - lo) // 2
        c = jnp.sum((bits <= mid).astype(jnp.int32))
        ge = c >= k
        return jnp.where(ge, lo, mid + 1), jnp.where(ge, mid, hi)

    lo, _ = jax.lax.fori_loop(
        0, 31, step, (jnp.int32(0), jnp.int32(_F32_INF_BITS))
    )
    t = jax.lax.bitcast_convert_type(lo, jnp.float32)
    lt = arr < t
    c_lt = jnp.sum(lt.astype(jnp.int32))
    slog = jnp.sum(jnp.where(lt, jnp.log(jnp.where(lt, arr, 1.0)), 0.0))
    return t, c_lt, slog


def _loss_body(d_ref, gt_ref, out_ref, negv_ref, cnt_ref, fac_ref):
    i = pl.program_id(0)

    @pl.when(i == 0)
    def _init():
        cnt_ref[0] = 0
        cnt_ref[1] = 0
        fac_ref[0] = 0.0
        fac_ref[1] = 0.0

    gxmin = gt_ref[:, 1:2]
    gymin = gt_ref[:, 2:3]
    gw = gt_ref[:, 3:4]
    gh = gt_ref[:, 4:5]
    gxmax = gxmin + gw
    gymax = gymin + gh
    gcx = gxmin + gw * 0.5
    gcy = gymin + gh * 0.5

    pbd = [d_ref[j : j + 1, :] for j in range(4)]
    cf0 = d_ref[4:5, :]
    cf1 = d_ref[5:6, :]
    acx = d_ref[6:7, :]
    acy = d_ref[7:8, :]
    aw = d_ref[8:9, :]
    ah = d_ref[9:10, :]
    axmin = d_ref[10:11, :]
    aymin = d_ref[11:12, :]
    axmax = d_ref[12:13, :]
    aymax = d_ref[13:14, :]
    valid = d_ref[14:15, :] < 0.5

    iw = jnp.maximum(jnp.minimum(axmax, gxmax) - jnp.maximum(axmin, gxmin), 0.0)
    ih = jnp.maximum(jnp.minimum(aymax, gymax) - jnp.maximum(aymin, gymin), 0.0)
    inter = iw * ih
    area_a = (axmax - axmin) * (aymax - aymin)
    area_b = gw * gh
    union = (area_a + area_b) - inter
    pos = ((inter + inter) >= union) & valid
    posf = pos.astype(jnp.float32)
    pos_per_anchor = jnp.sum(posf, axis=0, keepdims=True)
    neg_row = (pos_per_anchor == 0.0) & valid

    inv_aw = (1.0 / _VAR_X) / aw
    inv_ah = (1.0 / _VAR_Y) / ah
    law = jnp.log(aw) * (1.0 / _VAR_W)
    lah = jnp.log(ah) * (1.0 / _VAR_H)
    lgw = jnp.log(gw) * (1.0 / _VAR_W)
    lgh = jnp.log(gh) * (1.0 / _VAR_H)

    def _sl1(d):
        ad = jnp.abs(d)
        m = jnp.minimum(ad, 1.0)
        return m * (ad - 0.5 * m)

    s = _sl1(pbd[0] - (gcx - acx) * inv_aw)
    s = s + _sl1(pbd[1] - (gcy - acy) * inv_ah)
    s = s + _sl1(pbd[2] - (lgw - law))
    s = s + _sl1(pbd[3] - (lgh - lah))
    loc = jnp.sum(s * posf)

    cnt_ref[0] += jnp.sum(pos_per_anchor).astype(jnp.int32)
    cnt_ref[1] += jnp.sum(neg_row.astype(jnp.int32))
    fac_ref[0] += loc
    fac_ref[1] += jnp.sum(pos_per_anchor * jnp.log(cf0))

    negv_ref[pl.ds(i, 1), :] = jnp.where(neg_row, cf1, jnp.inf)

    @pl.when(i == _NB - 1)
    def _finalize():
        num_pos = cnt_ref[0]
        num_neg = cnt_ref[1]
        loc_loss = fac_ref[0]
        pos_cf_sum = fac_ref[1]

        neg_arr = negv_ref[:, :]
        c05 = jnp.sum((neg_arr < _MAX_BACK_CF).astype(jnp.int32))
        n_hard = jnp.minimum(jnp.maximum(num_pos * _NEG2POS, _MIN_NEG), num_neg)
        n_m = jnp.minimum(n_hard, c05)
        t, c_lt, slog = _kth_smallest_stats(neg_arr, n_m)
        t_safe = jnp.where(n_m > 0, t, 1.0)
        s_mined = slog + (n_m - c_lt).astype(jnp.float32) * jnp.log(t_safe)
        neg_cf_loss = jnp.where(
            n_m == 0,
            jnp.float32(0.0),
            -s_mined / jnp.maximum(n_m, 1).astype(jnp.float32) * _NEG_LAMBDA,
        )
        num_pos_f = jnp.maximum(num_pos, 1).astype(jnp.float32)
        loss = (
            _ALPHA * loc_loss / num_pos_f - pos_cf_sum / num_pos_f + neg_cf_loss
        )
        out_ref[:, :] = jnp.broadcast_to(loss, (1, 1))

        @pl.when(num_pos == 0)
        def _no_positives():
            t0, c0, slog0 = _kth_smallest_stats(neg_arr, _MIN_NEG)
            s0 = slog0 + (_MIN_NEG - c0).astype(jnp.float32) * jnp.log(t0)
            out_ref[:, :] = jnp.broadcast_to(
                -s0 / float(_MIN_NEG) * _NEG_LAMBDA, (1, 1)
            )


def kernel(pred_box_delt, pred_CF, GT_box_wh, Anchor_box_wh, Anchor_box_xy):
    stacked = jnp.concatenate(
        [pred_box_delt, pred_CF, Anchor_box_wh, Anchor_box_xy], axis=1
    )
    stacked = jnp.pad(stacked, ((0, 0), (0, 2)), constant_values=0.0)
    stacked = jnp.pad(stacked, ((0, _NPAD - _N), (0, 0)), constant_values=1.0)
    data = stacked.T
    out = pl.pallas_call(
        _loss_body,
        grid=(_NB,),
        in_specs=[
            pl.BlockSpec((16, _B), lambda i: (0, i)),
            pl.BlockSpec((_G, 5), lambda i: (0, 0)),
        ],
        out_specs=pl.BlockSpec((1, 1), lambda i: (0, 0)),
        out_shape=jax.ShapeDtypeStruct((1, 1), jnp.float32),
        scratch_shapes=[
            pltpu.VMEM((_NB, _B), jnp.float32),
            pltpu.SMEM((2,), jnp.int32),
            pltpu.SMEM((2,), jnp.float32),
        ],
    )(data, GT_box_wh)
    return out[0, 0]


# speedup vs baseline: 2.1653x; 1.0209x over previous
import jax
import jax.numpy as jnp
from jax.experimental import pallas as pl
from jax.experimental.pallas import tpu as pltpu

_N = 20000
_G = 64
_B = 2560
_NB = 8
_NPAD = _B * _NB

_VAR_X = 0.1
_VAR_Y = 0.1
_VAR_W = 0.2
_VAR_H = 0.2
_ALPHA = 1.0
_THR = 0.5
_NEG2POS = 6
_MIN_NEG = 10
_MAX_BACK_CF = 0.5
_NEG_LAMBDA = 1.0

_F32_INF_BITS = 0x7F800000


def _kth_smallest_stats(arr, k):
    bits = jax.lax.bitcast_convert_type(arr, jnp.int32)

    def step(_, lohi):
        lo, hi = lohi
        mid = lo + (hi - lo) // 2
        c = jnp.sum((bits <= mid).astype(jnp.int32))
        ge = c >= k
        return jnp.where(ge, lo, mid + 1), jnp.where(ge, mid, hi)

    lo, _ = jax.lax.fori_loop(
        0, 31, step, (jnp.int32(0), jnp.int32(_F32_INF_BITS))
    )
    t = jax.lax.bitcast_convert_type(lo, jnp.float32)
    lt = arr < t
    c_lt = jnp.sum(lt.astype(jnp.int32))
    slog = jnp.sum(jnp.where(lt, jnp.log(jnp.where(lt, arr, 1.0)), 0.0))
    return t, c_lt, slog


def _loss_body(d_ref, gt_ref, out_ref, negv_ref, cnt_ref, fac_ref):
    i = pl.program_id(0)

    @pl.when(i == 0)
    def _init():
        cnt_ref[0] = 0
        cnt_ref[1] = 0
        fac_ref[0] = 0.0
        fac_ref[1] = 0.0

    gxmin = gt_ref[:, 1:2]
    gymin = gt_ref[:, 2:3]
    gw = gt_ref[:, 3:4]
    gh = gt_ref[:, 4:5]
    gxmax = gxmin + gw
    gymax = gymin + gh
    gcx = gxmin + gw * 0.5
    gcy = gymin + gh * 0.5

    pbd = [d_ref[j : j + 1, :] for j in range(4)]
    cf0 = d_ref[4:5, :]
    cf1 = d_ref[5:6, :]
    acx = d_ref[6:7, :]
    acy = d_ref[7:8, :]
    aw = d_ref[8:9, :]
    ah = d_ref[9:10, :]
    axmin = d_ref[10:11, :]
    aymin = d_ref[11:12, :]
    axmax = d_ref[12:13, :]
    aymax = d_ref[13:14, :]
    valid = d_ref[14:15, :] < 0.5

    iw = jnp.maximum(jnp.minimum(axmax, gxmax) - jnp.maximum(axmin, gxmin), 0.0)
    ih = jnp.maximum(jnp.minimum(aymax, gymax) - jnp.maximum(aymin, gymin), 0.0)
    inter = iw * ih
    area_a = (axmax - axmin) * (aymax - aymin)
    area_b = gw * gh
    union = (area_a + area_b) - inter
    pos = ((inter + inter) >= union) & valid
    posf = pos.astype(jnp.float32)
    pos_per_anchor = jnp.sum(posf, axis=0, keepdims=True)
    neg_row = (pos_per_anchor == 0.0) & valid

    inv_aw = (1.0 / _VAR_X) / aw
    inv_ah = (1.0 / _VAR_Y) / ah
    law = jnp.log(aw) * (1.0 / _VAR_W)
    lah = jnp.log(ah) * (1.0 / _VAR_H)
    lgw = jnp.log(gw) * (1.0 / _VAR_W)
    lgh = jnp.log(gh) * (1.0 / _VAR_H)

    def _sl1(d):
        ad = jnp.abs(d)
        m = jnp.minimum(ad, 1.0)
        return m * (ad - 0.5 * m)

    s = _sl1(pbd[0] - (gcx - acx) * inv_aw)
    s = s + _sl1(pbd[1] - (gcy - acy) * inv_ah)
    s = s + _sl1(pbd[2] - (lgw - law))
    s = s + _sl1(pbd[3] - (lgh - lah))
    loc = jnp.sum(s * posf)

    cnt_ref[0] += jnp.sum(pos_per_anchor).astype(jnp.int32)
    cnt_ref[1] += jnp.sum(neg_row.astype(jnp.int32))
    fac_ref[0] += loc
    fac_ref[1] += jnp.sum(pos_per_anchor * jnp.log(cf0))

    negv_ref[pl.ds(i, 1), :] = jnp.where(neg_row, cf1, jnp.inf)

    @pl.when(i == _NB - 1)
    def _finalize():
        num_pos = cnt_ref[0]
        num_neg = cnt_ref[1]
        loc_loss = fac_ref[0]
        pos_cf_sum = fac_ref[1]

        neg_arr = negv_ref[:, :]
        c05 = jnp.sum((neg_arr < _MAX_BACK_CF).astype(jnp.int32))
        n_hard = jnp.minimum(jnp.maximum(num_pos * _NEG2POS, _MIN_NEG), num_neg)
        n_m = jnp.minimum(n_hard, c05)
        t, c_lt, slog = _kth_smallest_stats(neg_arr, n_m)
        t_safe = jnp.where(n_m > 0, t, 1.0)
        s_mined = slog + (n_m - c_lt).astype(jnp.float32) * jnp.log(t_safe)
        neg_cf_loss = jnp.where(
            n_m == 0,
            jnp.float32(0.0),
            -s_mined / jnp.maximum(n_m, 1).astype(jnp.float32) * _NEG_LAMBDA,
        )
        num_pos_f = jnp.maximum(num_pos, 1).astype(jnp.float32)
        loss = (
            _ALPHA * loc_loss / num_pos_f - pos_cf_sum / num_pos_f + neg_cf_loss
        )
        out_ref[:, :] = jnp.broadcast_to(loss, (1, 1))

        @pl.when(num_pos == 0)
        def _no_positives():
            t0, c0, slog0 = _kth_smallest_stats(neg_arr, _MIN_NEG)
            s0 = slog0 + (_MIN_NEG - c0).astype(jnp.float32) * jnp.log(t0)
            out_ref[:, :] = jnp.broadcast_to(
                -s0 / float(_MIN_NEG) * _NEG_LAMBDA, (1, 1)
            )


def kernel(pred_box_delt, pred_CF, GT_box_wh, Anchor_box_wh, Anchor_box_xy):
    stacked = jnp.concatenate(
        [pred_box_delt, pred_CF, Anchor_box_wh, Anchor_box_xy], axis=1
    )
    stacked = jnp.pad(stacked, ((0, 0), (0, 2)), constant_values=0.0)
    stacked = jnp.pad(stacked, ((0, _NPAD - _N), (0, 0)), constant_values=1.0)
    data = stacked.T
    out = pl.pallas_call(
        _loss_body,
        grid=(_NB,),
        in_specs=[
            pl.BlockSpec((16, _B), lambda i: (0, i)),
            pl.BlockSpec((_G, 5), lambda i: (0, 0)),
        ],
        out_specs=pl.BlockSpec((1, 1), lambda i: (0, 0)),
        out_shape=jax.ShapeDtypeStruct((1, 1), jnp.float32),
        scratch_shapes=[
            pltpu.VMEM((_NB, _B), jnp.float32),
            pltpu.SMEM((2,), jnp.int32),
            pltpu.SMEM((2,), jnp.float32),
        ],
    )(data, GT_box_wh)
    return out[0, 0]


# speedup vs baseline: 2.3465x; 1.0837x over previous
import jax
import jax.numpy as jnp
from jax.experimental import pallas as pl
from jax.experimental.pallas import tpu as pltpu

_N = 20000
_G = 64
_B = 2560
_NB = 8
_NPAD = _B * _NB

_VAR_X = 0.1
_VAR_Y = 0.1
_VAR_W = 0.2
_VAR_H = 0.2
_ALPHA = 1.0
_THR = 0.5
_NEG2POS = 6
_MIN_NEG = 10
_MAX_BACK_CF = 0.5
_NEG_LAMBDA = 1.0

_F32_INF_BITS = 0x7F800000


def _kth_smallest_stats(arr, k):
    bits = jax.lax.bitcast_convert_type(arr, jnp.int32)

    lo = jnp.int32(0)
    for shift in (27, 23, 19, 15, 11, 7, 3, 0):
        b = jax.lax.shift_right_arithmetic(bits - lo, shift)
        j_star = jnp.int32(0)
        for j in range(16):
            c_j = jnp.sum((b <= j).astype(jnp.int32))
            j_star = j_star + (c_j < k).astype(jnp.int32)
        lo = lo + jax.lax.shift_left(j_star, shift)
    t = jax.lax.bitcast_convert_type(lo, jnp.float32)
    lt = arr < t
    c_lt = jnp.sum(lt.astype(jnp.int32))
    slog = jnp.sum(jnp.where(lt, jnp.log(jnp.where(lt, arr, 1.0)), 0.0))
    return t, c_lt, slog


def _loss_body(d_ref, gt_ref, out_ref, negv_ref, cnt_ref, fac_ref):
    i = pl.program_id(0)

    @pl.when(i == 0)
    def _init():
        cnt_ref[0] = 0
        cnt_ref[1] = 0
        fac_ref[0] = 0.0
        fac_ref[1] = 0.0

    gxmin = gt_ref[:, 1:2]
    gymin = gt_ref[:, 2:3]
    gw = gt_ref[:, 3:4]
    gh = gt_ref[:, 4:5]
    gxmax = gxmin + gw
    gymax = gymin + gh
    gcx = gxmin + gw * 0.5
    gcy = gymin + gh * 0.5

    pbd = [d_ref[j : j + 1, :] for j in range(4)]
    cf0 = d_ref[4:5, :]
    cf1 = d_ref[5:6, :]
    acx = d_ref[6:7, :]
    acy = d_ref[7:8, :]
    aw = d_ref[8:9, :]
    ah = d_ref[9:10, :]
    axmin = d_ref[10:11, :]
    aymin = d_ref[11:12, :]
    axmax = d_ref[12:13, :]
    aymax = d_ref[13:14, :]
    valid = d_ref[14:15, :] < 0.5

    iw = jnp.maximum(jnp.minimum(axmax, gxmax) - jnp.maximum(axmin, gxmin), 0.0)
    ih = jnp.maximum(jnp.minimum(aymax, gymax) - jnp.maximum(aymin, gymin), 0.0)
    inter = iw * ih
    area_a = (axmax - axmin) * (aymax - aymin)
    area_b = gw * gh
    union = (area_a + area_b) - inter
    pos = ((inter + inter) >= union) & valid
    posf = pos.astype(jnp.float32)
    pos_per_anchor = jnp.sum(posf, axis=0, keepdims=True)
    neg_row = (pos_per_anchor == 0.0) & valid

    inv_aw = (1.0 / _VAR_X) / aw
    inv_ah = (1.0 / _VAR_Y) / ah
    law = jnp.log(aw) * (1.0 / _VAR_W)
    lah = jnp.log(ah) * (1.0 / _VAR_H)
    lgw = jnp.log(gw) * (1.0 / _VAR_W)
    lgh = jnp.log(gh) * (1.0 / _VAR_H)

    def _sl1(d):
        ad = jnp.abs(d)
        m = jnp.minimum(ad, 1.0)
        return m * (ad - 0.5 * m)

    s = _sl1(pbd[0] - (gcx - acx) * inv_aw)
    s = s + _sl1(pbd[1] - (gcy - acy) * inv_ah)
    s = s + _sl1(pbd[2] - (lgw - law))
    s = s + _sl1(pbd[3] - (lgh - lah))
    loc = jnp.sum(s * posf)

    cnt_ref[0] += jnp.sum(pos_per_anchor).astype(jnp.int32)
    cnt_ref[1] += jnp.sum(neg_row.astype(jnp.int32))
    fac_ref[0] += loc
    fac_ref[1] += jnp.sum(pos_per_anchor * jnp.log(cf0))

    negv_ref[pl.ds(i, 1), :] = jnp.where(neg_row, cf1, jnp.inf)

    @pl.when(i == _NB - 1)
    def _finalize():
        num_pos = cnt_ref[0]
        num_neg = cnt_ref[1]
        loc_loss = fac_ref[0]
        pos_cf_sum = fac_ref[1]

        neg_arr = negv_ref[:, :]
        c05 = jnp.sum((neg_arr < _MAX_BACK_CF).astype(jnp.int32))
        n_hard = jnp.minimum(jnp.maximum(num_pos * _NEG2POS, _MIN_NEG), num_neg)
        n_m = jnp.minimum(n_hard, c05)
        t, c_lt, slog = _kth_smallest_stats(neg_arr, n_m)
        t_safe = jnp.where(n_m > 0, t, 1.0)
        s_mined = slog + (n_m - c_lt).astype(jnp.float32) * jnp.log(t_safe)
        neg_cf_loss = jnp.where(
            n_m == 0,
            jnp.float32(0.0),
            -s_mined / jnp.maximum(n_m, 1).astype(jnp.float32) * _NEG_LAMBDA,
        )
        num_pos_f = jnp.maximum(num_pos, 1).astype(jnp.float32)
        loss = (
            _ALPHA * loc_loss / num_pos_f - pos_cf_sum / num_pos_f + neg_cf_loss
        )
        out_ref[:, :] = jnp.broadcast_to(loss, (1, 1))

        @pl.when(num_pos == 0)
        def _no_positives():
            t0, c0, slog0 = _kth_smallest_stats(neg_arr, _MIN_NEG)
            s0 = slog0 + (_MIN_NEG - c0).astype(jnp.float32) * jnp.log(t0)
            out_ref[:, :] = jnp.broadcast_to(
                -s0 / float(_MIN_NEG) * _NEG_LAMBDA, (1, 1)
            )


def kernel(pred_box_delt, pred_CF, GT_box_wh, Anchor_box_wh, Anchor_box_xy):
    stacked = jnp.concatenate(
        [pred_box_delt, pred_CF, Anchor_box_wh, Anchor_box_xy], axis=1
    )
    stacked = jnp.pad(stacked, ((0, 0), (0, 2)), constant_values=0.0)
    stacked = jnp.pad(stacked, ((0, _NPAD - _N), (0, 0)), constant_values=1.0)
    data = stacked.T
    out = pl.pallas_call(
        _loss_body,
        grid=(_NB,),
        in_specs=[
            pl.BlockSpec((16, _B), lambda i: (0, i)),
            pl.BlockSpec((_G, 5), lambda i: (0, 0)),
        ],
        out_specs=pl.BlockSpec((1, 1), lambda i: (0, 0)),
        out_shape=jax.ShapeDtypeStruct((1, 1), jnp.float32),
        scratch_shapes=[
            pltpu.VMEM((_NB, _B), jnp.float32),
            pltpu.SMEM((2,), jnp.int32),
            pltpu.SMEM((2,), jnp.float32),
        ],
    )(data, GT_box_wh)
    return out[0, 0]


# speedup vs baseline: 2.4213x; 1.0319x over previous
import jax
import jax.numpy as jnp
from jax.experimental import pallas as pl
from jax.experimental.pallas import tpu as pltpu

_N = 20000
_G = 64
_B = 2560
_NB = 8
_NPAD = _B * _NB

_VAR_X = 0.1
_VAR_Y = 0.1
_VAR_W = 0.2
_VAR_H = 0.2
_ALPHA = 1.0
_THR = 0.5
_NEG2POS = 6
_MIN_NEG = 10
_MAX_BACK_CF = 0.5
_NEG_LAMBDA = 1.0

_F32_INF_BITS = 0x7F800000


def _kth_smallest_stats(arr, k):
    bits = jax.lax.bitcast_convert_type(arr, jnp.int32)

    lo = jnp.int32(0)
    for shift in (27, 23, 19, 15, 11, 7, 3, 0):
        b = jax.lax.shift_right_arithmetic(bits - lo, shift)
        j_star = jnp.int32(0)
        for j in range(16):
            c_j = jnp.sum((b <= j).astype(jnp.int32))
            j_star = j_star + (c_j < k).astype(jnp.int32)
        lo = lo + jax.lax.shift_left(j_star, shift)
    t = jax.lax.bitcast_convert_type(lo, jnp.float32)
    lt = arr < t
    c_lt = jnp.sum(lt.astype(jnp.int32))
    slog = jnp.sum(jnp.where(lt, jnp.log(jnp.where(lt, arr, 1.0)), 0.0))
    return t, c_lt, slog


def _loss_body(d_ref, gt_ref, out_ref, negv_ref, cnt_ref, fac_ref):
    i = pl.program_id(0)

    @pl.when(i == 0)
    def _init():
        cnt_ref[0] = 0
        cnt_ref[1] = 0
        fac_ref[0] = 0.0
        fac_ref[1] = 0.0

    gxmin = gt_ref[:, 1:2]
    gymin = gt_ref[:, 2:3]
    gw = gt_ref[:, 3:4]
    gh = gt_ref[:, 4:5]
    gxmax = gxmin + gw
    gymax = gymin + gh
    gcx = gxmin + gw * 0.5
    gcy = gymin + gh * 0.5

    pbd = [d_ref[j : j + 1, :] for j in range(4)]
    cf0 = d_ref[4:5, :]
    cf1 = d_ref[5:6, :]
    acx = d_ref[6:7, :]
    acy = d_ref[7:8, :]
    aw = d_ref[8:9, :]
    ah = d_ref[9:10, :]
    axmin = d_ref[10:11, :]
    aymin = d_ref[11:12, :]
    axmax = d_ref[12:13, :]
    aymax = d_ref[13:14, :]
    valid = d_ref[14:15, :] < 0.5

    iw = jnp.maximum(jnp.minimum(axmax, gxmax) - jnp.maximum(axmin, gxmin), 0.0)
    ih = jnp.maximum(jnp.minimum(aymax, gymax) - jnp.maximum(aymin, gymin), 0.0)
    inter = iw * ih
    area_a = (axmax - axmin) * (aymax - aymin)
    area_b = gw * gh
    union = (area_a + area_b) - inter
    pos = (inter + inter) >= union
    posf = pos.astype(jnp.float32)
    pos_per_anchor = jnp.sum(posf, axis=0, keepdims=True)
    neg_row = (pos_per_anchor == 0.0) & valid

    inv_aw = (1.0 / _VAR_X) / aw
    inv_ah = (1.0 / _VAR_Y) / ah
    law = jnp.log(aw) * (1.0 / _VAR_W)
    lah = jnp.log(ah) * (1.0 / _VAR_H)
    lgw = jnp.log(gw) * (1.0 / _VAR_W)
    lgh = jnp.log(gh) * (1.0 / _VAR_H)

    def _sl1(d):
        ad = jnp.abs(d)
        m = jnp.minimum(ad, 1.0)
        return m * (ad - 0.5 * m)

    s = _sl1(pbd[0] - (gcx - acx) * inv_aw)
    s = s + _sl1(pbd[1] - (gcy - acy) * inv_ah)
    s = s + _sl1(pbd[2] - (lgw - law))
    s = s + _sl1(pbd[3] - (lgh - lah))
    loc = jnp.sum(s * posf)

    cnt_ref[0] += jnp.sum(pos_per_anchor).astype(jnp.int32)
    cnt_ref[1] += jnp.sum(neg_row.astype(jnp.int32))
    fac_ref[0] += loc
    fac_ref[1] += jnp.sum(pos_per_anchor * jnp.log(cf0))

    negv_ref[pl.ds(i, 1), :] = jnp.where(neg_row, cf1, jnp.inf)

    @pl.when(i == _NB - 1)
    def _finalize():
        num_pos = cnt_ref[0]
        num_neg = cnt_ref[1]
        loc_loss = fac_ref[0]
        pos_cf_sum = fac_ref[1]

        neg_arr = negv_ref[:, :]
        c05 = jnp.sum((neg_arr < _MAX_BACK_CF).astype(jnp.int32))
        n_hard = jnp.minimum(jnp.maximum(num_pos * _NEG2POS, _MIN_NEG), num_neg)
        n_m = jnp.minimum(n_hard, c05)
        t, c_lt, slog = _kth_smallest_stats(neg_arr, n_m)
        t_safe = jnp.where(n_m > 0, t, 1.0)
        s_mined = slog + (n_m - c_lt).astype(jnp.float32) * jnp.log(t_safe)
        neg_cf_loss = jnp.where(
            n_m == 0,
            jnp.float32(0.0),
            -s_mined / jnp.maximum(n_m, 1).astype(jnp.float32) * _NEG_LAMBDA,
        )
        num_pos_f = jnp.maximum(num_pos, 1).astype(jnp.float32)
        loss = (
            _ALPHA * loc_loss / num_pos_f - pos_cf_sum / num_pos_f + neg_cf_loss
        )
        out_ref[:, :] = jnp.broadcast_to(loss, (1, 1))

        @pl.when(num_pos == 0)
        def _no_positives():
            t0, c0, slog0 = _kth_smallest_stats(neg_arr, _MIN_NEG)
            s0 = slog0 + (_MIN_NEG - c0).astype(jnp.float32) * jnp.log(t0)
            out_ref[:, :] = jnp.broadcast_to(
                -s0 / float(_MIN_NEG) * _NEG_LAMBDA, (1, 1)
            )


def kernel(pred_box_delt, pred_CF, GT_box_wh, Anchor_box_wh, Anchor_box_xy):
    stacked = jnp.concatenate(
        [pred_box_delt, pred_CF, Anchor_box_wh, Anchor_box_xy], axis=1
    )
    stacked = jnp.pad(stacked, ((0, 0), (0, 2)), constant_values=0.0)
    stacked = jnp.pad(stacked, ((0, _NPAD - _N), (0, 0)), constant_values=1.0)
    data = stacked.T
    out = pl.pallas_call(
        _loss_body,
        grid=(_NB,),
        in_specs=[
            pl.BlockSpec((16, _B), lambda i: (0, i)),
            pl.BlockSpec((_G, 5), lambda i: (0, 0)),
        ],
        out_specs=pl.BlockSpec((1, 1), lambda i: (0, 0)),
        out_shape=jax.ShapeDtypeStruct((1, 1), jnp.float32),
        scratch_shapes=[
            pltpu.VMEM((_NB, _B), jnp.float32),
            pltpu.SMEM((2,), jnp.int32),
            pltpu.SMEM((2,), jnp.float32),
        ],
    )(data, GT_box_wh)
    return out[0, 0]


# speedup vs baseline: 2.5230x; 1.0420x over previous
import jax
import jax.numpy as jnp
from jax.experimental import pallas as pl
from jax.experimental.pallas import tpu as pltpu

_N = 20000
_G = 64
_B = 2560
_NB = 8
_NPAD = _B * _NB

_VAR_X = 0.1
_VAR_Y = 0.1
_VAR_W = 0.2
_VAR_H = 0.2
_ALPHA = 1.0
_THR = 0.5
_NEG2POS = 6
_MIN_NEG = 10
_MAX_BACK_CF = 0.5
_NEG_LAMBDA = 1.0

_F32_INF_BITS = 0x7F800000


def _kth_smallest_stats(arr, k):
    bits = jax.lax.bitcast_convert_type(arr, jnp.int32)

    lo = jnp.int32(0)
    for shift in (27, 23, 19, 15, 11, 7, 3, 0):
        b = jax.lax.shift_right_arithmetic(bits - lo, shift)
        j_star = jnp.int32(0)
        for j in range(16):
            c_j = jnp.sum((b <= j).astype(jnp.int32))
            j_star = j_star + (c_j < k).astype(jnp.int32)
        lo = lo + jax.lax.shift_left(j_star, shift)
    t = jax.lax.bitcast_convert_type(lo, jnp.float32)
    lt = arr < t
    c_lt = jnp.sum(lt.astype(jnp.int32))
    slog = jnp.sum(jnp.where(lt, jnp.log(jnp.where(lt, arr, 1.0)), 0.0))
    return t, c_lt, slog


def _loss_body(d_ref, gt_ref, out_ref, negv_ref, cnt_ref, fac_ref):
    i = pl.program_id(0)

    @pl.when(i == 0)
    def _init():
        cnt_ref[0] = 0
        cnt_ref[1] = 0
        fac_ref[0] = 0.0
        fac_ref[1] = 0.0

    gxmin = gt_ref[:, 1:2]
    gymin = gt_ref[:, 2:3]
    gw = gt_ref[:, 3:4]
    gh = gt_ref[:, 4:5]
    gxmax = gxmin + gw
    gymax = gymin + gh
    gcx = gxmin + gw * 0.5
    gcy = gymin + gh * 0.5

    pbd = [d_ref[j : j + 1, :] for j in range(4)]
    cf0 = d_ref[4:5, :]
    cf1 = d_ref[5:6, :]
    acx = d_ref[6:7, :]
    acy = d_ref[7:8, :]
    aw = d_ref[8:9, :]
    ah = d_ref[9:10, :]
    axmin = d_ref[10:11, :]
    aymin = d_ref[11:12, :]
    axmax = d_ref[12:13, :]
    aymax = d_ref[13:14, :]
    lane = jax.lax.broadcasted_iota(jnp.int32, (1, _B), 1)
    valid = (lane + i * _B) < _N

    iw = jnp.maximum(jnp.minimum(axmax, gxmax) - jnp.maximum(axmin, gxmin), 0.0)
    ih = jnp.maximum(jnp.minimum(aymax, gymax) - jnp.maximum(aymin, gymin), 0.0)
    inter = iw * ih
    area_a = (axmax - axmin) * (aymax - aymin)
    area_b = gw * gh
    union = (area_a + area_b) - inter
    pos = (inter + inter) >= union
    posf = pos.astype(jnp.float32)
    pos_per_anchor = jnp.sum(posf, axis=0, keepdims=True)
    neg_row = (pos_per_anchor == 0.0) & valid

    inv_aw = (1.0 / _VAR_X) / aw
    inv_ah = (1.0 / _VAR_Y) / ah
    law = jnp.log(aw) * (1.0 / _VAR_W)
    lah = jnp.log(ah) * (1.0 / _VAR_H)
    lgw = jnp.log(gw) * (1.0 / _VAR_W)
    lgh = jnp.log(gh) * (1.0 / _VAR_H)

    def _sl1(d):
        ad = jnp.abs(d)
        m = jnp.minimum(ad, 1.0)
        return m * (ad - 0.5 * m)

    s = _sl1(pbd[0] - (gcx - acx) * inv_aw)
    s = s + _sl1(pbd[1] - (gcy - acy) * inv_ah)
    s = s + _sl1(pbd[2] - (lgw - law))
    s = s + _sl1(pbd[3] - (lgh - lah))
    loc = jnp.sum(s * posf)

    cnt_ref[0] += jnp.sum(pos_per_anchor).astype(jnp.int32)
    cnt_ref[1] += jnp.sum(neg_row.astype(jnp.int32))
    fac_ref[0] += loc
    fac_ref[1] += jnp.sum(pos_per_anchor * jnp.log(cf0))

    negv_ref[pl.ds(i, 1), :] = jnp.where(neg_row, cf1, jnp.inf)

    @pl.when(i == _NB - 1)
    def _finalize():
        num_pos = cnt_ref[0]
        num_neg = cnt_ref[1]
        loc_loss = fac_ref[0]
        pos_cf_sum = fac_ref[1]

        neg_arr = negv_ref[:, :]
        c05 = jnp.sum((neg_arr < _MAX_BACK_CF).astype(jnp.int32))
        n_hard = jnp.minimum(jnp.maximum(num_pos * _NEG2POS, _MIN_NEG), num_neg)
        n_m = jnp.minimum(n_hard, c05)
        t, c_lt, slog = _kth_smallest_stats(neg_arr, n_m)
        t_safe = jnp.where(n_m > 0, t, 1.0)
        s_mined = slog + (n_m - c_lt).astype(jnp.float32) * jnp.log(t_safe)
        neg_cf_loss = jnp.where(
            n_m == 0,
            jnp.float32(0.0),
            -s_mined / jnp.maximum(n_m, 1).astype(jnp.float32) * _NEG_LAMBDA,
        )
        num_pos_f = jnp.maximum(num_pos, 1).astype(jnp.float32)
        loss = (
            _ALPHA * loc_loss / num_pos_f - pos_cf_sum / num_pos_f + neg_cf_loss
        )
        out_ref[:, :] = jnp.broadcast_to(loss, (1, 1))

        @pl.when(num_pos == 0)
        def _no_positives():
            t0, c0, slog0 = _kth_smallest_stats(neg_arr, _MIN_NEG)
            s0 = slog0 + (_MIN_NEG - c0).astype(jnp.float32) * jnp.log(t0)
            out_ref[:, :] = jnp.broadcast_to(
                -s0 / float(_MIN_NEG) * _NEG_LAMBDA, (1, 1)
            )


def kernel(pred_box_delt, pred_CF, GT_box_wh, Anchor_box_wh, Anchor_box_xy):
    stacked = jnp.concatenate(
        [pred_box_delt, pred_CF, Anchor_box_wh, Anchor_box_xy], axis=1
    )
    data = stacked.T
    data = jnp.pad(data, ((0, 0), (0, _NPAD - _N)), constant_values=1.0)
    out = pl.pallas_call(
        _loss_body,
        grid=(_NB,),
        in_specs=[
            pl.BlockSpec((14, _B), lambda i: (0, i)),
            pl.BlockSpec((_G, 5), lambda i: (0, 0)),
        ],
        out_specs=pl.BlockSpec((1, 1), lambda i: (0, 0)),
        out_shape=jax.ShapeDtypeStruct((1, 1), jnp.float32),
        scratch_shapes=[
            pltpu.VMEM((_NB, _B), jnp.float32),
            pltpu.SMEM((2,), jnp.int32),
            pltpu.SMEM((2,), jnp.float32),
        ],
    )(data, GT_box_wh)
    return out[0, 0]


# speedup vs baseline: 2.8827x; 1.1425x over previous
import jax
import jax.numpy as jnp
from jax.experimental import pallas as pl
from jax.experimental.pallas import tpu as pltpu

_N = 20000
_G = 64
_B = 2560
_NB = 8
_NPAD = _B * _NB

_VAR_X = 0.1
_VAR_Y = 0.1
_VAR_W = 0.2
_VAR_H = 0.2
_ALPHA = 1.0
_THR = 0.5
_NEG2POS = 6
_MIN_NEG = 10
_MAX_BACK_CF = 0.5
_NEG_LAMBDA = 1.0

_F32_INF_BITS = 0x7F800000


def _kth_smallest_stats(arr, k):
    bits = jax.lax.bitcast_convert_type(arr, jnp.int32)

    lo = jnp.int32(0)
    for shift in (27, 23, 19, 15, 11, 7, 3, 0):
        b = jax.lax.shift_right_arithmetic(bits - lo, shift)
        j_star = jnp.int32(0)
        for j in range(16):
            c_j = jnp.sum((b <= j).astype(jnp.int32))
            j_star = j_star + (c_j < k).astype(jnp.int32)
        lo = lo + jax.lax.shift_left(j_star, shift)
    t = jax.lax.bitcast_convert_type(lo, jnp.float32)
    lt = arr < t
    c_lt = jnp.sum(lt.astype(jnp.int32))
    slog = jnp.sum(jnp.where(lt, jnp.log(jnp.where(lt, arr, 1.0)), 0.0))
    return t, c_lt, slog


def _loss_body(d_ref, gt_ref, out_ref, negv_ref, acc_ref):
    i = pl.program_id(0)

    @pl.when(i == 0)
    def _init():
        acc_ref[:, :] = jnp.zeros((4, _B), jnp.float32)

    gxmin = gt_ref[:, 1:2]
    gymin = gt_ref[:, 2:3]
    gw = gt_ref[:, 3:4]
    gh = gt_ref[:, 4:5]
    gxmax = gxmin + gw
    gymax = gymin + gh
    gcx = gxmin + gw * 0.5
    gcy = gymin + gh * 0.5

    pbd = [d_ref[j : j + 1, :] for j in range(4)]
    cf0 = d_ref[4:5, :]
    cf1 = d_ref[5:6, :]
    acx = d_ref[6:7, :]
    acy = d_ref[7:8, :]
    aw = d_ref[8:9, :]
    ah = d_ref[9:10, :]
    axmin = d_ref[10:11, :]
    aymin = d_ref[11:12, :]
    axmax = d_ref[12:13, :]
    aymax = d_ref[13:14, :]
    lane = jax.lax.broadcasted_iota(jnp.int32, (1, _B), 1)
    valid = (lane + i * _B) < _N

    iw = jnp.maximum(jnp.minimum(axmax, gxmax) - jnp.maximum(axmin, gxmin), 0.0)
    ih = jnp.maximum(jnp.minimum(aymax, gymax) - jnp.maximum(aymin, gymin), 0.0)
    inter = iw * ih
    area_a = (axmax - axmin) * (aymax - aymin)
    area_b = gw * gh
    union = (area_a + area_b) - inter
    pos = (inter + inter) >= union
    posf = pos.astype(jnp.float32)
    pos_per_anchor = jnp.sum(posf, axis=0, keepdims=True)
    neg_row = (pos_per_anchor == 0.0) & valid

    inv_aw = (1.0 / _VAR_X) / aw
    inv_ah = (1.0 / _VAR_Y) / ah
    law = jnp.log(aw) * (1.0 / _VAR_W)
    lah = jnp.log(ah) * (1.0 / _VAR_H)
    lgw = jnp.log(gw) * (1.0 / _VAR_W)
    lgh = jnp.log(gh) * (1.0 / _VAR_H)

    def _sl1(d):
        ad = jnp.abs(d)
        m = jnp.minimum(ad, 1.0)
        return m * (ad - 0.5 * m)

    s = _sl1(pbd[0] - (gcx - acx) * inv_aw)
    s = s + _sl1(pbd[1] - (gcy - acy) * inv_ah)
    s = s + _sl1(pbd[2] - (lgw - law))
    s = s + _sl1(pbd[3] - (lgh - lah))
    loc = jnp.sum(s * posf, axis=0, keepdims=True)

    acc_ref[0:1, :] += pos_per_anchor
    acc_ref[1:2, :] += neg_row.astype(jnp.float32)
    acc_ref[2:3, :] += loc
    acc_ref[3:4, :] += pos_per_anchor * jnp.log(cf0)

    negv_ref[pl.ds(i, 1), :] = jnp.where(neg_row, cf1, jnp.inf)

    @pl.when(i == _NB - 1)
    def _finalize():
        num_pos = jnp.sum(acc_ref[0:1, :]).astype(jnp.int32)
        num_neg = jnp.sum(acc_ref[1:2, :]).astype(jnp.int32)
        loc_loss = jnp.sum(acc_ref[2:3, :])
        pos_cf_sum = jnp.sum(acc_ref[3:4, :])

        neg_arr = negv_ref[:, :]
        c05 = jnp.sum((neg_arr < _MAX_BACK_CF).astype(jnp.int32))
        n_hard = jnp.minimum(jnp.maximum(num_pos * _NEG2POS, _MIN_NEG), num_neg)
        n_m = jnp.minimum(n_hard, c05)
        t, c_lt, slog = _kth_smallest_stats(neg_arr, n_m)
        t_safe = jnp.where(n_m > 0, t, 1.0)
        s_mined = slog + (n_m - c_lt).astype(jnp.float32) * jnp.log(t_safe)
        neg_cf_loss = jnp.where(
            n_m == 0,
            jnp.float32(0.0),
            -s_mined / jnp.maximum(n_m, 1).astype(jnp.float32) * _NEG_LAMBDA,
        )
        num_pos_f = jnp.maximum(num_pos, 1).astype(jnp.float32)
        loss = (
            _ALPHA * loc_loss / num_pos_f - pos_cf_sum / num_pos_f + neg_cf_loss
        )
        out_ref[:, :] = jnp.broadcast_to(loss, (1, 1))

        @pl.when(num_pos == 0)
        def _no_positives():
            t0, c0, slog0 = _kth_smallest_stats(neg_arr, _MIN_NEG)
            s0 = slog0 + (_MIN_NEG - c0).astype(jnp.float32) * jnp.log(t0)
            out_ref[:, :] = jnp.broadcast_to(
                -s0 / float(_MIN_NEG) * _NEG_LAMBDA, (1, 1)
            )


def kernel(pred_box_delt, pred_CF, GT_box_wh, Anchor_box_wh, Anchor_box_xy):
    stacked = jnp.concatenate(
        [pred_box_delt, pred_CF, Anchor_box_wh, Anchor_box_xy], axis=1
    )
    data = stacked.T
    data = jnp.pad(data, ((0, 0), (0, _NPAD - _N)), constant_values=1.0)
    out = pl.pallas_call(
        _loss_body,
        grid=(_NB,),
        in_specs=[
            pl.BlockSpec((14, _B), lambda i: (0, i)),
            pl.BlockSpec((_G, 5), lambda i: (0, 0)),
        ],
        out_specs=pl.BlockSpec((1, 1), lambda i: (0, 0)),
        out_shape=jax.ShapeDtypeStruct((1, 1), jnp.float32),
        scratch_shapes=[
            pltpu.VMEM((_NB, _B), jnp.float32),
            pltpu.VMEM((4, _B), jnp.float32),
        ],
    )(data, GT_box_wh)
    return out[0, 0]


# speedup vs baseline: 3.0368x; 1.0535x over previous
import jax
import jax.numpy as jnp
from jax.experimental import pallas as pl
from jax.experimental.pallas import tpu as pltpu

_N = 20000
_G = 64
_B = 2560
_NB = 8
_NPAD = _B * _NB

_VAR_X = 0.1
_VAR_Y = 0.1
_VAR_W = 0.2
_VAR_H = 0.2
_ALPHA = 1.0
_THR = 0.5
_NEG2POS = 6
_MIN_NEG = 10
_MAX_BACK_CF = 0.5
_NEG_LAMBDA = 1.0

_F32_INF_BITS = 0x7F800000


def _kth_smallest_stats(arr, k):
    bits = jax.lax.bitcast_convert_type(arr, jnp.int32)

    lo = jnp.int32(0)
    for shift in (27, 23, 19, 15, 11, 7, 3, 0):
        b = jax.lax.shift_right_arithmetic(bits - lo, shift)
        j_star = jnp.int32(0)
        for j in range(16):
            c_j = jnp.sum((b <= j).astype(jnp.int32))
            j_star = j_star + (c_j < k).astype(jnp.int32)
        lo = lo + jax.lax.shift_left(j_star, shift)
    t = jax.lax.bitcast_convert_type(lo, jnp.float32)
    lt = arr < t
    c_lt = jnp.sum(lt.astype(jnp.int32))
    slog = jnp.sum(jnp.where(lt, jnp.log(jnp.where(lt, arr, 1.0)), 0.0))
    return t, c_lt, slog


def _loss_body(d_ref, gt_ref, out_ref, negv_ref, acc_ref):
    i = pl.program_id(0)

    @pl.when(i == 0)
    def _init():
        acc_ref[:, :] = jnp.zeros((4, _B), jnp.float32)

    gxmin = gt_ref[:, 1:2]
    gymin = gt_ref[:, 2:3]
    gw = gt_ref[:, 3:4]
    gh = gt_ref[:, 4:5]
    gxmax = gxmin + gw
    gymax = gymin + gh
    gcx = gxmin + gw * 0.5
    gcy = gymin + gh * 0.5

    pbd = [d_ref[j : j + 1, :] for j in range(4)]
    cf0 = d_ref[4:5, :]
    cf1 = d_ref[5:6, :]
    acx = d_ref[6:7, :]
    acy = d_ref[7:8, :]
    aw = d_ref[8:9, :]
    ah = d_ref[9:10, :]
    axmin = d_ref[10:11, :]
    aymin = d_ref[11:12, :]
    axmax = d_ref[12:13, :]
    aymax = d_ref[13:14, :]
    lane = jax.lax.broadcasted_iota(jnp.int32, (1, _B), 1)
    valid = (lane + i * _B) < _N

    iw = jnp.maximum(jnp.minimum(axmax, gxmax) - jnp.maximum(axmin, gxmin), 0.0)
    ih = jnp.maximum(jnp.minimum(aymax, gymax) - jnp.maximum(aymin, gymin), 0.0)
    inter = iw * ih
    area_a = (axmax - axmin) * (aymax - aymin)
    area_b = gw * gh
    union = (area_a + area_b) - inter
    pos = (inter + inter) >= union
    posf = pos.astype(jnp.float32)
    pos_per_anchor = jnp.sum(posf, axis=0, keepdims=True)
    neg_row = (pos_per_anchor == 0.0) & valid

    inv_aw = (1.0 / _VAR_X) / aw
    inv_ah = (1.0 / _VAR_Y) / ah
    law = jnp.log(aw) * (1.0 / _VAR_W)
    lah = jnp.log(ah) * (1.0 / _VAR_H)
    lgw = jnp.log(gw) * (1.0 / _VAR_W)
    lgh = jnp.log(gh) * (1.0 / _VAR_H)

    def _sl1(d):
        ad = jnp.abs(d)
        m = jnp.minimum(ad, 1.0)
        return m * (ad - 0.5 * m)

    p0 = pbd[0] + acx * inv_aw
    p1 = pbd[1] + acy * inv_ah
    p2 = pbd[2] + law
    p3 = pbd[3] + lah
    s = _sl1(p0 - gcx * inv_aw)
    s = s + _sl1(p1 - gcy * inv_ah)
    s = s + _sl1(p2 - lgw)
    s = s + _sl1(p3 - lgh)
    loc = jnp.sum(s * posf, axis=0, keepdims=True)

    acc_ref[0:1, :] += pos_per_anchor
    acc_ref[1:2, :] += neg_row.astype(jnp.float32)
    acc_ref[2:3, :] += loc
    acc_ref[3:4, :] += pos_per_anchor * jnp.log(cf0)

    negv_ref[pl.ds(i, 1), :] = jnp.where(neg_row, cf1, jnp.inf)

    @pl.when(i == _NB - 1)
    def _finalize():
        num_pos = jnp.sum(acc_ref[0:1, :]).astype(jnp.int32)
        num_neg = jnp.sum(acc_ref[1:2, :]).astype(jnp.int32)
        loc_loss = jnp.sum(acc_ref[2:3, :])
        pos_cf_sum = jnp.sum(acc_ref[3:4, :])

        neg_arr = negv_ref[:, :]
        c05 = jnp.sum((neg_arr < _MAX_BACK_CF).astype(jnp.int32))
        n_hard = jnp.minimum(jnp.maximum(num_pos * _NEG2POS, _MIN_NEG), num_neg)
        n_m = jnp.minimum(n_hard, c05)
        t, c_lt, slog = _kth_smallest_stats(neg_arr, n_m)
        t_safe = jnp.where(n_m > 0, t, 1.0)
        s_mined = slog + (n_m - c_lt).astype(jnp.float32) * jnp.log(t_safe)
        neg_cf_loss = jnp.where(
            n_m == 0,
            jnp.float32(0.0),
            -s_mined / jnp.maximum(n_m, 1).astype(jnp.float32) * _NEG_LAMBDA,
        )
        num_pos_f = jnp.maximum(num_pos, 1).astype(jnp.float32)
        loss = (
            _ALPHA * loc_loss / num_pos_f - pos_cf_sum / num_pos_f + neg_cf_loss
        )
        out_ref[:, :] = jnp.broadcast_to(loss, (1, 1))

        @pl.when(num_pos == 0)
        def _no_positives():
            t0, c0, slog0 = _kth_smallest_stats(neg_arr, _MIN_NEG)
            s0 = slog0 + (_MIN_NEG - c0).astype(jnp.float32) * jnp.log(t0)
            out_ref[:, :] = jnp.broadcast_to(
                -s0 / float(_MIN_NEG) * _NEG_LAMBDA, (1, 1)
            )


def kernel(pred_box_delt, pred_CF, GT_box_wh, Anchor_box_wh, Anchor_box_xy):
    stacked = jnp.concatenate(
        [pred_box_delt, pred_CF, Anchor_box_wh, Anchor_box_xy], axis=1
    )
    data = stacked.T
    data = jnp.pad(data, ((0, 0), (0, _NPAD - _N)), constant_values=1.0)
    out = pl.pallas_call(
        _loss_body,
        grid=(_NB,),
        in_specs=[
            pl.BlockSpec((14, _B), lambda i: (0, i)),
            pl.BlockSpec((_G, 5), lambda i: (0, 0)),
        ],
        out_specs=pl.BlockSpec((1, 1), lambda i: (0, 0)),
        out_shape=jax.ShapeDtypeStruct((1, 1), jnp.float32),
        scratch_shapes=[
            pltpu.VMEM((_NB, _B), jnp.float32),
            pltpu.VMEM((4, _B), jnp.float32),
        ],
    )(data, GT_box_wh)
    return out[0, 0]


# speedup vs baseline: 3.0620x; 1.0083x over previous
import jax
import jax.numpy as jnp
from jax.experimental import pallas as pl
from jax.experimental.pallas import tpu as pltpu

_N = 20000
_G = 64
_B = 2560
_NB = 8
_NPAD = _B * _NB

_VAR_X = 0.1
_VAR_Y = 0.1
_VAR_W = 0.2
_VAR_H = 0.2
_ALPHA = 1.0
_THR = 0.5
_NEG2POS = 6
_MIN_NEG = 10
_MAX_BACK_CF = 0.5
_NEG_LAMBDA = 1.0

_F32_INF_BITS = 0x7F800000


def _kth_smallest_stats(arr, k):
    bits = jax.lax.bitcast_convert_type(arr, jnp.int32)

    lo = jnp.int32(0)
    for shift in (27, 23, 19, 15, 11, 7, 3, 0):
        b = jax.lax.shift_right_arithmetic(bits - lo, shift)
        j_star = jnp.int32(0)
        for j in range(16):
            c_j = jnp.sum((b <= j).astype(jnp.int32))
            j_star = j_star + (c_j < k).astype(jnp.int32)
        lo = lo + jax.lax.shift_left(j_star, shift)
    t = jax.lax.bitcast_convert_type(lo, jnp.float32)
    lt = arr < t
    c_lt = jnp.sum(lt.astype(jnp.int32))
    slog = jnp.sum(jnp.where(lt, jnp.log(jnp.where(lt, arr, 1.0)), 0.0))
    return t, c_lt, slog


def _loss_body(d_ref, gt_ref, out_ref, negv_ref, acc_ref):
    i = pl.program_id(0)

    @pl.when(i == 0)
    def _init():
        acc_ref[:, :] = jnp.zeros((4, _B), jnp.float32)

    gxmin = gt_ref[:, 1:2]
    gymin = gt_ref[:, 2:3]
    gw = gt_ref[:, 3:4]
    gh = gt_ref[:, 4:5]
    gxmax = gxmin + gw
    gymax = gymin + gh
    gcx = gxmin + gw * 0.5
    gcy = gymin + gh * 0.5

    pbd = [d_ref[j : j + 1, :] for j in range(4)]
    cf0 = d_ref[4:5, :]
    cf1 = d_ref[5:6, :]
    acx = d_ref[6:7, :]
    acy = d_ref[7:8, :]
    aw = d_ref[8:9, :]
    ah = d_ref[9:10, :]
    axmin = d_ref[10:11, :]
    aymin = d_ref[11:12, :]
    axmax = d_ref[12:13, :]
    aymax = d_ref[13:14, :]
    lane = jax.lax.broadcasted_iota(jnp.int32, (1, _B), 1)
    valid = (lane + i * _B) < _N

    iw = jnp.maximum(jnp.minimum(axmax, gxmax) - jnp.maximum(axmin, gxmin), 0.0)
    ih = jnp.maximum(jnp.minimum(aymax, gymax) - jnp.maximum(aymin, gymin), 0.0)
    inter = iw * ih
    area_a = (axmax - axmin) * (aymax - aymin)
    area_b = gw * gh
    union = (area_a + area_b) - inter
    pos = (inter / union) >= _THR
    posf = pos.astype(jnp.float32)
    pos_per_anchor = jnp.sum(posf, axis=0, keepdims=True)
    neg_row = (pos_per_anchor == 0.0) & valid

    inv_aw = (1.0 / _VAR_X) / aw
    inv_ah = (1.0 / _VAR_Y) / ah
    law = jnp.log(aw) * (1.0 / _VAR_W)
    lah = jnp.log(ah) * (1.0 / _VAR_H)
    lgw = jnp.log(gw) * (1.0 / _VAR_W)
    lgh = jnp.log(gh) * (1.0 / _VAR_H)

    def _sl1(d):
        ad = jnp.abs(d)
        m = jnp.minimum(ad, 1.0)
        return m * (ad - 0.5 * m)

    p0 = pbd[0] + acx * inv_aw
    p1 = pbd[1] + acy * inv_ah
    p2 = pbd[2] + law
    p3 = pbd[3] + lah
    s = _sl1(p0 - gcx * inv_aw)
    s = s + _sl1(p1 - gcy * inv_ah)
    s = s + _sl1(p2 - lgw)
    s = s + _sl1(p3 - lgh)
    loc = jnp.sum(s * posf, axis=0, keepdims=True)

    acc_ref[0:1, :] += pos_per_anchor
    acc_ref[1:2, :] += neg_row.astype(jnp.float32)
    acc_ref[2:3, :] += loc
    acc_ref[3:4, :] += pos_per_anchor * jnp.log(cf0)

    negv_ref[pl.ds(i, 1), :] = jnp.where(neg_row, cf1, jnp.inf)

    @pl.when(i == _NB - 1)
    def _finalize():
        num_pos = jnp.sum(acc_ref[0:1, :]).astype(jnp.int32)
        num_neg = jnp.sum(acc_ref[1:2, :]).astype(jnp.int32)
        loc_loss = jnp.sum(acc_ref[2:3, :])
        pos_cf_sum = jnp.sum(acc_ref[3:4, :])

        neg_arr = negv_ref[:, :]
        c05 = jnp.sum((neg_arr < _MAX_BACK_CF).astype(jnp.int32))
        n_hard = jnp.minimum(jnp.maximum(num_pos * _NEG2POS, _MIN_NEG), num_neg)
        n_m = jnp.minimum(n_hard, c05)
        t, c_lt, slog = _kth_smallest_stats(neg_arr, n_m)
        t_safe = jnp.where(n_m > 0, t, 1.0)
        s_mined = slog + (n_m - c_lt).astype(jnp.float32) * jnp.log(t_safe)
        neg_cf_loss = jnp.where(
            n_m == 0,
            jnp.float32(0.0),
            -s_mined / jnp.maximum(n_m, 1).astype(jnp.float32) * _NEG_LAMBDA,
        )
        num_pos_f = jnp.maximum(num_pos, 1).astype(jnp.float32)
        loss = (
            _ALPHA * loc_loss / num_pos_f - pos_cf_sum / num_pos_f + neg_cf_loss
        )
        out_ref[:, :] = jnp.broadcast_to(loss, (1, 1))

        @pl.when(num_pos == 0)
        def _no_positives():
            t0, c0, slog0 = _kth_smallest_stats(neg_arr, _MIN_NEG)
            s0 = slog0 + (_MIN_NEG - c0).astype(jnp.float32) * jnp.log(t0)
            out_ref[:, :] = jnp.broadcast_to(
                -s0 / float(_MIN_NEG) * _NEG_LAMBDA, (1, 1)
            )


def kernel(pred_box_delt, pred_CF, GT_box_wh, Anchor_box_wh, Anchor_box_xy):
    stacked = jnp.concatenate(
        [pred_box_delt, pred_CF, Anchor_box_wh, Anchor_box_xy], axis=1
    )
    data = stacked.T
    data = jnp.pad(data, ((0, 0), (0, _NPAD - _N)), constant_values=1.0)
    out = pl.pallas_call(
        _loss_body,
        grid=(_NB,),
        in_specs=[
            pl.BlockSpec((14, _B), lambda i: (0, i)),
            pl.BlockSpec((_G, 5), lambda i: (0, 0)),
        ],
        out_specs=pl.BlockSpec((1, 1), lambda i: (0, 0)),
        out_shape=jax.ShapeDtypeStruct((1, 1), jnp.float32),
        scratch_shapes=[
            pltpu.VMEM((_NB, _B), jnp.float32),
            pltpu.VMEM((4, _B), jnp.float32),
        ],
    )(data, GT_box_wh)
    return out[0, 0]
